# baseline (speedup 1.0000x reference)
import math

import jax
import jax.numpy as jnp
import numpy as np
from jax import lax
from jax.experimental import pallas as pl
from jax.experimental.pallas import tpu as pltpu
from jax.experimental.pallas import tpu_sc as plsc

_NUM_HEADS = 16
_L = 1024
_GRID = 32
_SLAB_COLS = 2016
_SLAB_PAD = 2048
_LANES = 16


def _piecewise_index(rp: np.ndarray) -> np.ndarray:
    alpha, beta, gamma = 1.9, 3.8, 15.2
    rp = rp.astype(np.float32)
    rp_abs = np.abs(rp)
    mask = rp_abs <= alpha
    safe_abs = np.where(mask, 1.0, rp_abs)
    y = np.sign(rp) * np.minimum(
        np.round(alpha + np.log(safe_abs / alpha) / math.log(gamma / alpha)
                 * (beta - alpha)), beta)
    return np.where(mask, np.round(rp), y).astype(np.int32)


def _build_slab_index_map() -> np.ndarray:
    f = _piecewise_index(np.arange(-31, 32)) + 3
    xi = np.arange(_GRID)
    m = np.arange(2 * _GRID - 1)
    fm = f[62 - m]
    fx = f[(xi[:, None] - xi[None, :]) + 31]
    full = (fm[None, :, None] * 7 + fx[:, None, :]).reshape(_GRID, _SLAB_COLS)
    padded = np.zeros((_GRID, _SLAB_PAD), np.int32)
    padded[:, :_SLAB_COLS] = full
    return np.ascontiguousarray(
        padded.reshape(2, 16, _SLAB_PAD).astype(np.int32))


_SLAB_MAP = _build_slab_index_map()


def _sc_gather_kernel(table_hbm, map_hbm, wall_hbm, tab_v, map_v, w_v):
    c = lax.axis_index("c")
    s = lax.axis_index("s")
    pltpu.sync_copy(table_hbm, tab_v)
    pltpu.sync_copy(map_hbm.at[c], map_v)
    hoff = jnp.broadcast_to(s * 64, (_LANES,)).astype(jnp.int32)

    n_chunks = _SLAB_PAD // _LANES

    @plsc.parallel_loop(0, n_chunks, unroll=8)
    def chunk_body(j):
        off = j * _LANES
        for r in range(16):
            idx = map_v[r, pl.ds(off, _LANES)] + hoff
            w_v[r, pl.ds(off, _LANES)] = plsc.load_gather(tab_v, [idx])
    pltpu.sync_copy(w_v, wall_hbm.at[s, pl.ds(c * 16, 16), :])


def _tc_expand_kernel(w_ref, out_ref):
    w = w_ref[0]
    for yi in range(_GRID):
        start = (31 - yi) * _GRID
        out_ref[0, 0, yi * _GRID:(yi + 1) * _GRID, :] = (
            w[:, start:start + _L])


def kernel(x, lookup_table_bias):
    del x
    table = jnp.pad(lookup_table_bias, ((0, 0), (0, 15))).reshape(-1)
    slab_map = jnp.asarray(_SLAB_MAP)

    mesh = plsc.VectorSubcoreMesh(core_axis_name="c", subcore_axis_name="s")
    gather = pl.kernel(
        _sc_gather_kernel,
        out_type=jax.ShapeDtypeStruct((_NUM_HEADS, _GRID, _SLAB_PAD),
                                      jnp.float32),
        mesh=mesh,
        scratch_types=[
            pltpu.VMEM((_NUM_HEADS * 64,), jnp.float32),
            pltpu.VMEM((16, _SLAB_PAD), jnp.int32),
            pltpu.VMEM((16, _SLAB_PAD), jnp.float32),
        ],
        compiler_params=pltpu.CompilerParams(
            use_tc_tiling_on_sc=True, needs_layout_passes=False),
    )
    wall = gather(table, slab_map)

    expand = pl.pallas_call(
        _tc_expand_kernel,
        grid=(_NUM_HEADS,),
        in_specs=[pl.BlockSpec((1, _GRID, _SLAB_PAD), lambda h: (h, 0, 0))],
        out_specs=pl.BlockSpec((1, 1, _L, _L), lambda h: (0, h, 0, 0)),
        out_shape=jax.ShapeDtypeStruct((1, _NUM_HEADS, _L, _L), jnp.float32),
    )
    return expand(wall)


# speedup vs baseline: 1.0071x; 1.0071x over previous
import math

import jax
import jax.numpy as jnp
import numpy as np
from jax import lax
from jax.experimental import pallas as pl
from jax.experimental.pallas import tpu as pltpu
from jax.experimental.pallas import tpu_sc as plsc

_NUM_HEADS = 16
_L = 1024
_GRID = 32
_SLAB_COLS = 2016
_SLAB_PAD = 2048
_LANES = 16


def _piecewise_index(rp: np.ndarray) -> np.ndarray:
    alpha, beta, gamma = 1.9, 3.8, 15.2
    rp = rp.astype(np.float32)
    rp_abs = np.abs(rp)
    mask = rp_abs <= alpha
    safe_abs = np.where(mask, 1.0, rp_abs)
    y = np.sign(rp) * np.minimum(
        np.round(alpha + np.log(safe_abs / alpha) / math.log(gamma / alpha)
                 * (beta - alpha)), beta)
    return np.where(mask, np.round(rp), y).astype(np.int32)


def _build_slab_index_map() -> np.ndarray:
    f = _piecewise_index(np.arange(-31, 32)) + 3
    xi = np.arange(_GRID)
    m = np.arange(2 * _GRID - 1)
    fm = f[62 - m]
    fx = f[(xi[:, None] - xi[None, :]) + 31]
    full = (fm[None, :, None] * 7 + fx[:, None, :]).reshape(_GRID, _SLAB_COLS)
    padded = np.zeros((_GRID, _SLAB_PAD), np.int32)
    padded[:, :_SLAB_COLS] = full
    return np.ascontiguousarray(
        padded.reshape(2, 16, _SLAB_PAD).astype(np.int32))


_SLAB_MAP = _build_slab_index_map()


def _sc_gather_kernel(table_hbm, map_hbm, wall_hbm, tab_v, map_v, w_v):
    c = lax.axis_index("c")
    s = lax.axis_index("s")
    pltpu.sync_copy(table_hbm, tab_v)
    pltpu.sync_copy(map_hbm.at[c], map_v)
    hoff = jnp.broadcast_to(s * 64, (_LANES,)).astype(jnp.int32)

    n_chunks = _SLAB_PAD // _LANES

    @plsc.parallel_loop(0, n_chunks, unroll=4)
    def chunk_body(j):
        off = j * _LANES
        for r in range(16):
            idx = map_v[r, pl.ds(off, _LANES)] + hoff
            w_v[r, pl.ds(off, _LANES)] = plsc.load_gather(tab_v, [idx])
    pltpu.sync_copy(w_v, wall_hbm.at[s, pl.ds(c * 16, 16), :])


def _tc_expand_kernel(w_ref, out_ref):
    w = w_ref[0]
    for yi in range(_GRID):
        start = (31 - yi) * _GRID
        out_ref[0, 0, yi * _GRID:(yi + 1) * _GRID, :] = (
            w[:, start:start + _L])


def kernel(x, lookup_table_bias):
    del x
    table = jnp.pad(lookup_table_bias, ((0, 0), (0, 15))).reshape(-1)
    slab_map = jnp.asarray(_SLAB_MAP)

    mesh = plsc.VectorSubcoreMesh(core_axis_name="c", subcore_axis_name="s")
    gather = pl.kernel(
        _sc_gather_kernel,
        out_type=jax.ShapeDtypeStruct((_NUM_HEADS, _GRID, _SLAB_PAD),
                                      jnp.float32),
        mesh=mesh,
        scratch_types=[
            pltpu.VMEM((_NUM_HEADS * 64,), jnp.float32),
            pltpu.VMEM((16, _SLAB_PAD), jnp.int32),
            pltpu.VMEM((16, _SLAB_PAD), jnp.float32),
        ],
        compiler_params=pltpu.CompilerParams(
            use_tc_tiling_on_sc=True, needs_layout_passes=False),
    )
    wall = gather(table, slab_map)

    expand = pl.pallas_call(
        _tc_expand_kernel,
        grid=(_NUM_HEADS,),
        in_specs=[pl.BlockSpec((1, _GRID, _SLAB_PAD), lambda h: (h, 0, 0))],
        out_specs=pl.BlockSpec((1, 1, _L, _L), lambda h: (0, h, 0, 0)),
        out_shape=jax.ShapeDtypeStruct((1, _NUM_HEADS, _L, _L), jnp.float32),
    )
    return expand(wall)


# speedup vs baseline: 1.0111x; 1.0040x over previous
import math

import jax
import jax.numpy as jnp
import numpy as np
from jax import lax
from jax.experimental import pallas as pl
from jax.experimental.pallas import tpu as pltpu
from jax.experimental.pallas import tpu_sc as plsc

_NUM_HEADS = 16
_L = 1024
_GRID = 32
_SLAB_COLS = 2016
_SLAB_PAD = 2048
_LANES = 16


def _piecewise_index(rp: np.ndarray) -> np.ndarray:
    alpha, beta, gamma = 1.9, 3.8, 15.2
    rp = rp.astype(np.float32)
    rp_abs = np.abs(rp)
    mask = rp_abs <= alpha
    safe_abs = np.where(mask, 1.0, rp_abs)
    y = np.sign(rp) * np.minimum(
        np.round(alpha + np.log(safe_abs / alpha) / math.log(gamma / alpha)
                 * (beta - alpha)), beta)
    return np.where(mask, np.round(rp), y).astype(np.int32)


def _build_slab_index_map() -> np.ndarray:
    f = _piecewise_index(np.arange(-31, 32)) + 3
    xi = np.arange(_GRID)
    m = np.arange(2 * _GRID - 1)
    fm = f[62 - m]
    fx = f[(xi[:, None] - xi[None, :]) + 31]
    full = (fm[None, :, None] * 7 + fx[:, None, :]).reshape(_GRID, _SLAB_COLS)
    padded = np.zeros((_GRID, _SLAB_PAD), np.int32)
    padded[:, :_SLAB_COLS] = full
    return np.ascontiguousarray(
        padded.reshape(2, 16, _SLAB_PAD).astype(np.int32))


_SLAB_MAP = _build_slab_index_map()


def _sc_gather_kernel(table_hbm, map_hbm, wall_hbm, tab_v, map_v, w_v):
    c = lax.axis_index("c")
    s = lax.axis_index("s")
    pltpu.sync_copy(table_hbm, tab_v)
    pltpu.sync_copy(map_hbm.at[c], map_v)
    hoff = jnp.broadcast_to(s * 49, (_LANES,)).astype(jnp.int32)

    n_chunks = _SLAB_PAD // _LANES

    @plsc.parallel_loop(0, n_chunks, unroll=4)
    def chunk_body(j):
        off = j * _LANES
        for r in range(16):
            idx = map_v[r, pl.ds(off, _LANES)] + hoff
            w_v[r, pl.ds(off, _LANES)] = plsc.load_gather(tab_v, [idx])
    pltpu.sync_copy(w_v, wall_hbm.at[s, pl.ds(c * 16, 16), :])


def _tc_expand_kernel(w_ref, out_ref):
    w = w_ref[0]
    for yi in range(_GRID):
        start = (31 - yi) * _GRID
        out_ref[0, 0, yi * _GRID:(yi + 1) * _GRID, :] = (
            w[:, start:start + _L])


def kernel(x, lookup_table_bias):
    del x
    table = lookup_table_bias.reshape(-1)
    slab_map = jnp.asarray(_SLAB_MAP)

    mesh = plsc.VectorSubcoreMesh(core_axis_name="c", subcore_axis_name="s")
    gather = pl.kernel(
        _sc_gather_kernel,
        out_type=jax.ShapeDtypeStruct((_NUM_HEADS, _GRID, _SLAB_PAD),
                                      jnp.float32),
        mesh=mesh,
        scratch_types=[
            pltpu.VMEM((_NUM_HEADS * 49,), jnp.float32),
            pltpu.VMEM((16, _SLAB_PAD), jnp.int32),
            pltpu.VMEM((16, _SLAB_PAD), jnp.float32),
        ],
        compiler_params=pltpu.CompilerParams(
            use_tc_tiling_on_sc=True, needs_layout_passes=False),
    )
    wall = gather(table, slab_map)

    expand = pl.pallas_call(
        _tc_expand_kernel,
        grid=(_NUM_HEADS,),
        in_specs=[pl.BlockSpec((1, _GRID, _SLAB_PAD), lambda h: (h, 0, 0))],
        out_specs=pl.BlockSpec((1, 1, _L, _L), lambda h: (0, h, 0, 0)),
        out_shape=jax.ShapeDtypeStruct((1, _NUM_HEADS, _L, _L), jnp.float32),
    )
    return expand(wall)
